# Initial kernel scaffold; baseline (speedup 1.0000x reference)
import jax
import jax.numpy as jnp
from jax.experimental import pallas as pl


def kernel(x, offsets, w1, b1, w_hm, b_hm, w2, b2, w_wh, b_wh, w3, b3, w_reg, b_reg):
    raise NotImplementedError("write your pallas kernel here")


# speedup vs baseline: 1.3032x; 1.3032x over previous
import jax
import jax.numpy as jnp
from jax.experimental import pallas as pl

_H = 128
_W = 128
_WP = _W + 2
_HP = _H + 2
_NFLAT = _H * _WP
_FLAT = 17024
_NC = 1664
_NCHUNKS = _NFLAT // _NC


def _center_body(xf_ref, wt_ref, bcat_ref, wblk_ref, bh_ref,
                 hm_ref, wh_ref, reg_ref):
    for j in range(_NCHUNKS):
        base = j * _NC
        acc = jnp.zeros((288, _NC), jnp.float32)
        for dy in range(3):
            for dx in range(3):
                t = dy * 3 + dx
                off = dy * _WP + dx
                xs = xf_ref[0, :, base + off:base + off + _NC]
                acc = acc + jax.lax.dot_general(
                    wt_ref[t], xs, (((1,), (0,)), ((), ())),
                    preferred_element_type=jnp.float32)
        y = jnp.maximum(acc + bcat_ref[...], 0.0)
        heads = jax.lax.dot_general(
            wblk_ref[...], y, (((1,), (0,)), ((), ())),
            preferred_element_type=jnp.float32) + bh_ref[...]
        hm_ref[0, :, base:base + _NC] = heads[0:80, :]
        whreg = jnp.maximum(heads[80:84, :], 0.0)
        wh_ref[0, :, base:base + _NC] = whreg[0:2, :]
        reg_ref[0, :, base:base + _NC] = whreg[2:4, :]


def _center_call(xf, wt, bcat, wblk, bh, nb, interpret=False):
    return pl.pallas_call(
        _center_body,
        grid=(nb,),
        in_specs=[
            pl.BlockSpec((1, 96, _FLAT), lambda i: (i, 0, 0)),
            pl.BlockSpec((9, 288, 96), lambda i: (0, 0, 0)),
            pl.BlockSpec((288, 1), lambda i: (0, 0)),
            pl.BlockSpec((84, 288), lambda i: (0, 0)),
            pl.BlockSpec((84, 1), lambda i: (0, 0)),
        ],
        out_specs=[
            pl.BlockSpec((1, 80, _NFLAT), lambda i: (i, 0, 0)),
            pl.BlockSpec((1, 2, _NFLAT), lambda i: (i, 0, 0)),
            pl.BlockSpec((1, 2, _NFLAT), lambda i: (i, 0, 0)),
        ],
        out_shape=[
            jax.ShapeDtypeStruct((nb, 80, _NFLAT), jnp.float32),
            jax.ShapeDtypeStruct((nb, 2, _NFLAT), jnp.float32),
            jax.ShapeDtypeStruct((nb, 2, _NFLAT), jnp.float32),
        ],
        interpret=interpret,
    )(xf, wt, bcat, wblk, bh)


def kernel(x, offsets, w1, b1, w_hm, b_hm, w2, b2, w_wh, b_wh, w3, b3,
           w_reg, b_reg):
    nb = x.shape[0]
    xp = jnp.pad(x, ((0, 0), (0, 0), (1, 1), (1, 1)))
    xf = xp.reshape(nb, 96, _HP * _WP)
    xf = jnp.pad(xf, ((0, 0), (0, 0), (0, _FLAT - _HP * _WP)))

    wcat = jnp.concatenate([w1, w2, w3], axis=0)
    wt = jnp.transpose(wcat, (2, 3, 0, 1)).reshape(9, 288, 96)
    bcat = jnp.concatenate([b1, b2, b3]).reshape(288, 1)
    wblk = jnp.zeros((84, 288), jnp.float32)
    wblk = wblk.at[0:80, 0:96].set(w_hm.reshape(80, 96))
    wblk = wblk.at[80:82, 96:192].set(w_wh.reshape(2, 96))
    wblk = wblk.at[82:84, 192:288].set(w_reg.reshape(2, 96))
    bh = jnp.concatenate([b_hm, b_wh, b_reg]).reshape(84, 1)

    hm_f, wh_f, reg_f = _center_call(xf, wt, bcat, wblk, bh, nb)

    hm = hm_f.reshape(nb, 80, _H, _WP)[:, :, :, :_W]
    wh = wh_f.reshape(nb, 2, _H, _WP)[:, :, :, :_W]
    reg = reg_f.reshape(nb, 2, _H, _WP)[:, :, :, :_W]
    return (hm, wh, reg, offsets)


# speedup vs baseline: 1.3384x; 1.0270x over previous
import jax
import jax.numpy as jnp
from jax.experimental import pallas as pl

_H = 128
_W = 128
_WP = _W + 2
_HP = _H + 2
_NFLAT = _H * _WP
_FLAT = 17024
_NC = 1664
_NCHUNKS = _NFLAT // _NC


def _center_body(xf_ref, wt_ref, bcat_ref, wblk_ref, bh_ref,
                 hm_ref, wh_ref, reg_ref):
    for j in range(_NCHUNKS):
        base = j * _NC
        acc = jnp.zeros((288, _NC), jnp.float32)
        for dy in range(3):
            for dx in range(3):
                t = dy * 3 + dx
                off = dy * _WP + dx
                xs = xf_ref[0, :, base + off:base + off + _NC]
                acc = acc + jax.lax.dot_general(
                    wt_ref[t], xs, (((1,), (0,)), ((), ())),
                    preferred_element_type=jnp.float32)
        y = jnp.maximum(acc + bcat_ref[...], 0.0).astype(jnp.bfloat16)
        heads = jax.lax.dot_general(
            wblk_ref[...], y, (((1,), (0,)), ((), ())),
            preferred_element_type=jnp.float32) + bh_ref[...]
        hm_ref[0, :, base:base + _NC] = heads[0:80, :]
        whreg = jnp.maximum(heads[80:84, :], 0.0)
        wh_ref[0, :, base:base + _NC] = whreg[0:2, :]
        reg_ref[0, :, base:base + _NC] = whreg[2:4, :]


def _center_call(xf, wt, bcat, wblk, bh, nb, interpret=False):
    return pl.pallas_call(
        _center_body,
        grid=(nb,),
        in_specs=[
            pl.BlockSpec((1, 96, _FLAT), lambda i: (i, 0, 0)),
            pl.BlockSpec((9, 288, 96), lambda i: (0, 0, 0)),
            pl.BlockSpec((288, 1), lambda i: (0, 0)),
            pl.BlockSpec((84, 288), lambda i: (0, 0)),
            pl.BlockSpec((84, 1), lambda i: (0, 0)),
        ],
        out_specs=[
            pl.BlockSpec((1, 80, _NFLAT), lambda i: (i, 0, 0)),
            pl.BlockSpec((1, 2, _NFLAT), lambda i: (i, 0, 0)),
            pl.BlockSpec((1, 2, _NFLAT), lambda i: (i, 0, 0)),
        ],
        out_shape=[
            jax.ShapeDtypeStruct((nb, 80, _NFLAT), jnp.float32),
            jax.ShapeDtypeStruct((nb, 2, _NFLAT), jnp.float32),
            jax.ShapeDtypeStruct((nb, 2, _NFLAT), jnp.float32),
        ],
        interpret=interpret,
    )(xf, wt, bcat, wblk, bh)


def kernel(x, offsets, w1, b1, w_hm, b_hm, w2, b2, w_wh, b_wh, w3, b3,
           w_reg, b_reg):
    nb = x.shape[0]
    xp = jnp.pad(x.astype(jnp.bfloat16), ((0, 0), (0, 0), (1, 1), (1, 1)))
    xf = xp.reshape(nb, 96, _HP * _WP)
    xf = jnp.pad(xf, ((0, 0), (0, 0), (0, _FLAT - _HP * _WP)))

    wcat = jnp.concatenate([w1, w2, w3], axis=0)
    wt = jnp.transpose(wcat, (2, 3, 0, 1)).reshape(9, 288, 96)
    wt = wt.astype(jnp.bfloat16)
    bcat = jnp.concatenate([b1, b2, b3]).reshape(288, 1)
    wblk = jnp.zeros((84, 288), jnp.float32)
    wblk = wblk.at[0:80, 0:96].set(w_hm.reshape(80, 96))
    wblk = wblk.at[80:82, 96:192].set(w_wh.reshape(2, 96))
    wblk = wblk.at[82:84, 192:288].set(w_reg.reshape(2, 96))
    wblk = wblk.astype(jnp.bfloat16)
    bh = jnp.concatenate([b_hm, b_wh, b_reg]).reshape(84, 1)

    hm_f, wh_f, reg_f = _center_call(xf, wt, bcat, wblk, bh, nb)

    hm = hm_f.reshape(nb, 80, _H, _WP)[:, :, :, :_W]
    wh = wh_f.reshape(nb, 2, _H, _WP)[:, :, :, :_W]
    reg = reg_f.reshape(nb, 2, _H, _WP)[:, :, :, :_W]
    return (hm, wh, reg, offsets)


# speedup vs baseline: 2.2006x; 1.6442x over previous
import jax
import jax.numpy as jnp
from jax.experimental import pallas as pl
from jax.experimental.pallas import tpu as pltpu

_H = 128
_W = 128
_WP = _W + 2
_HP = _H + 2
_NFLAT = _H * _WP
_FLAT = 17024
_NC = 1664
_NCHUNKS = _NFLAT // _NC


def _center_body(x_ref, wt_ref, bcat_ref, wblk_ref, bh_ref,
                 hm_ref, wh_ref, reg_ref, xs_scr, hd_scr):
    @pl.when(pl.program_id(0) == 0)
    def _():
        xs_scr[...] = jnp.zeros((96, _FLAT), jnp.bfloat16)

    for h in range(_H):
        dst = (h + 1) * _WP + 1
        xs_scr[:, dst:dst + _W] = x_ref[0, :, h * _W:(h + 1) * _W].astype(
            jnp.bfloat16)

    for j in range(_NCHUNKS):
        base = j * _NC
        acc = jnp.zeros((288, _NC), jnp.float32)
        for dy in range(3):
            for dx in range(3):
                t = dy * 3 + dx
                off = dy * _WP + dx
                xsl = xs_scr[:, base + off:base + off + _NC]
                acc = acc + jax.lax.dot_general(
                    wt_ref[t], xsl, (((1,), (0,)), ((), ())),
                    preferred_element_type=jnp.float32)
        y = jnp.maximum(acc + bcat_ref[...], 0.0).astype(jnp.bfloat16)
        hd_scr[:, base:base + _NC] = jax.lax.dot_general(
            wblk_ref[...], y, (((1,), (0,)), ((), ())),
            preferred_element_type=jnp.float32) + bh_ref[...]

    for h in range(_H):
        blk = hd_scr[:, h * _WP:h * _WP + _W]
        hm_ref[0, :, h, :] = blk[0:80, :]
        whreg = jnp.maximum(blk[80:84, :], 0.0)
        wh_ref[0, :, h, :] = whreg[0:2, :]
        reg_ref[0, :, h, :] = whreg[2:4, :]


def kernel(x, offsets, w1, b1, w_hm, b_hm, w2, b2, w_wh, b_wh, w3, b3,
           w_reg, b_reg):
    nb = x.shape[0]
    xf = x.reshape(nb, 96, _H * _W)

    wcat = jnp.concatenate([w1, w2, w3], axis=0)
    wt = jnp.transpose(wcat, (2, 3, 0, 1)).reshape(9, 288, 96)
    wt = wt.astype(jnp.bfloat16)
    bcat = jnp.concatenate([b1, b2, b3]).reshape(288, 1)
    wblk = jnp.zeros((84, 288), jnp.float32)
    wblk = wblk.at[0:80, 0:96].set(w_hm.reshape(80, 96))
    wblk = wblk.at[80:82, 96:192].set(w_wh.reshape(2, 96))
    wblk = wblk.at[82:84, 192:288].set(w_reg.reshape(2, 96))
    wblk = wblk.astype(jnp.bfloat16)
    bh = jnp.concatenate([b_hm, b_wh, b_reg]).reshape(84, 1)

    hm, wh, reg = pl.pallas_call(
        _center_body,
        grid=(nb,),
        in_specs=[
            pl.BlockSpec((1, 96, _H * _W), lambda i: (i, 0, 0)),
            pl.BlockSpec((9, 288, 96), lambda i: (0, 0, 0)),
            pl.BlockSpec((288, 1), lambda i: (0, 0)),
            pl.BlockSpec((84, 288), lambda i: (0, 0)),
            pl.BlockSpec((84, 1), lambda i: (0, 0)),
        ],
        out_specs=[
            pl.BlockSpec((1, 80, _H, _W), lambda i: (i, 0, 0, 0)),
            pl.BlockSpec((1, 2, _H, _W), lambda i: (i, 0, 0, 0)),
            pl.BlockSpec((1, 2, _H, _W), lambda i: (i, 0, 0, 0)),
        ],
        out_shape=[
            jax.ShapeDtypeStruct((nb, 80, _H, _W), jnp.float32),
            jax.ShapeDtypeStruct((nb, 2, _H, _W), jnp.float32),
            jax.ShapeDtypeStruct((nb, 2, _H, _W), jnp.float32),
        ],
        scratch_shapes=[
            pltpu.VMEM((96, _FLAT), jnp.bfloat16),
            pltpu.VMEM((84, _NFLAT), jnp.float32),
        ],
    )(xf, wt, bcat, wblk, bh)

    return (hm, wh, reg, offsets)
